# Initial kernel scaffold; baseline (speedup 1.0000x reference)
import jax
import jax.numpy as jnp
from jax.experimental import pallas as pl


def kernel(input1_features, input1_coords, input2_features, input2_coords):
    raise NotImplementedError("write your pallas kernel here")


# speedup vs baseline: 36.9869x; 36.9869x over previous
import functools

import jax
import jax.numpy as jnp
from jax import lax
from jax.experimental import pallas as pl
from jax.experimental.pallas import tpu as pltpu
from jax.experimental.pallas import tpu_sc as plsc

D = 32
W = 32768
G = 128
SH = W + 256
TRASH = W
NZCH = SH // G


def _extract(bounds_ref, j):
    v = bounds_ref[pl.ds(j, 16)]
    return v[0]


def _make_merge_kernel(npad, nblocks):
    mesh = plsc.VectorSubcoreMesh(core_axis_name="c", subcore_axis_name="s")

    @functools.partial(
        pl.kernel,
        out_type=jax.ShapeDtypeStruct((npad, D), jnp.float32),
        mesh=mesh,
        compiler_params=pltpu.CompilerParams(use_tc_tiling_on_sc=False),
        scratch_types=[
            pltpu.VMEM((48,), jnp.int32),
            pltpu.VMEM((G,), jnp.int32),
            pltpu.VMEM((G,), jnp.int32),
            pltpu.VMEM((G,), jnp.int32),
            pltpu.VMEM((G, D), jnp.float32),
            pltpu.VMEM((G, 16), jnp.float32),
            pltpu.VMEM((G, D), jnp.float32),
            pltpu.VMEM((G, 16), jnp.float32),
            pltpu.VMEM((G, D), jnp.float32),
            pltpu.VMEM((G, 16), jnp.float32),
            pltpu.VMEM((G, D), jnp.float32),
            pltpu.VMEM_SHARED((SH, D), jnp.float32),
            pltpu.VMEM_SHARED((SH, 16), jnp.float32),
            pltpu.SemaphoreType.DMA,
        ],
    )
    def merge(feat, perm, ranks, bounds, out,
              bounds_v, perm_v, rank_v, dest_v, rows_v, ones_v, zrows_v,
              zcnt_v, sstage_v, cstage_v, ostage_v, sums_sh, cnts_sh, sem):
        c = lax.axis_index("c")
        s = lax.axis_index("s")

        pltpu.sync_copy(bounds, bounds_v)

        ones16 = jnp.ones((16,), jnp.float32)
        zeros16 = jnp.zeros((16,), jnp.float32)

        def init_body(i, carry):
            ones_v[i] = ones16
            zcnt_v[i] = zeros16
            zrows_v[i, pl.ds(0, 16)] = zeros16
            zrows_v[i, pl.ds(16, 16)] = zeros16
            return carry
        lax.fori_loop(jnp.int32(0), jnp.int32(G), init_body, 0)

        nblk = (nblocks - c + 1) // 2

        def block_body(jb, carry):
            k = c + 2 * jb
            base = k * W

            def zero_body(i, zcarry):
                ch = s + i * 16

                @pl.when(ch < NZCH)
                def _():
                    r0 = ch * G
                    pltpu.sync_copy(zrows_v, sums_sh.at[pl.ds(r0, G)])
                    pltpu.sync_copy(zcnt_v, cnts_sh.at[pl.ds(r0, G)])
                return zcarry
            lax.fori_loop(jnp.int32(0), jnp.int32((NZCH + 15) // 16), zero_body, 0)
            plsc.subcore_barrier()

            lo = _extract(bounds_v, k)
            hi = _extract(bounds_v, k + 1)
            lo8 = (lo // 8) * 8
            nch = (hi - lo8 + (G - 1)) // G
            my = jnp.maximum((nch - s + 15) // 16, 0)

            def acc_body(i, acarry):
                m = s + i * 16
                start = lo8 + m * G
                pltpu.sync_copy(perm.at[pl.ds(start, G)], perm_v)
                pltpu.sync_copy(ranks.at[pl.ds(start, G)], rank_v)
                for q in range(G // 16):
                    r = rank_v[pl.ds(q * 16, 16)]
                    off = r - base
                    valid = (off >= 0) & (off < W)
                    dest_v[pl.ds(q * 16, 16)] = jnp.where(valid, off, TRASH)
                pltpu.async_copy(feat.at[perm_v], rows_v, sem).wait()
                pltpu.sync_copy(rows_v, sums_sh.at[dest_v], add=True)
                pltpu.sync_copy(ones_v, cnts_sh.at[dest_v], add=True)
                return acarry
            lax.fori_loop(jnp.int32(0), my, acc_body, 0)
            plsc.subcore_barrier()

            tbase = s * (W // 16)
            for sub in range((W // 16) // G):
                r0 = tbase + sub * G
                pltpu.sync_copy(sums_sh.at[pl.ds(r0, G)], sstage_v)
                pltpu.sync_copy(cnts_sh.at[pl.ds(r0, G)], cstage_v)

                def div_body(rr, dcarry):
                    d = jnp.maximum(cstage_v[rr], 1.0)
                    ostage_v[rr, pl.ds(0, 16)] = sstage_v[rr, pl.ds(0, 16)] / d
                    ostage_v[rr, pl.ds(16, 16)] = sstage_v[rr, pl.ds(16, 16)] / d
                    return dcarry
                lax.fori_loop(jnp.int32(0), jnp.int32(G), div_body, 0)
                pltpu.sync_copy(ostage_v, out.at[pl.ds(base + r0, G)])
            plsc.subcore_barrier()
            return carry

        lax.fori_loop(jnp.int32(0), nblk, block_body, 0)

    return merge


def kernel(input1_features, input1_coords, input2_features, input2_coords):
    S = jnp.int64(4096)
    batch_size = 1 + jnp.max(input1_coords[:, 3])
    b2 = input2_coords[:, 3] + batch_size
    k1 = ((input1_coords[:, 3] * S + input1_coords[:, 0]) * S
          + input1_coords[:, 1]) * S + input1_coords[:, 2]
    k2 = ((b2 * S + input2_coords[:, 0]) * S
          + input2_coords[:, 1]) * S + input2_coords[:, 2]
    keys = jnp.concatenate([k1, k2])
    feats = jnp.concatenate([input1_features, input2_features], axis=0)
    n = keys.shape[0]

    sk, perm = lax.sort_key_val(keys, jnp.arange(n, dtype=jnp.int32))
    neq = (sk[1:] != sk[:-1]).astype(jnp.int32)
    ranks = jnp.concatenate([jnp.zeros((1,), jnp.int32), jnp.cumsum(neq, dtype=jnp.int32)])

    nblocks = -(-n // W)
    npad = nblocks * W
    targets = jnp.arange(nblocks + 1, dtype=jnp.int32) * W
    bounds = jnp.searchsorted(ranks, targets, side='left').astype(jnp.int32)
    bounds = jnp.concatenate(
        [bounds, jnp.full((48 - (nblocks + 1),), jnp.int32(n))])

    perm_p = jnp.concatenate([perm, jnp.zeros((G,), jnp.int32)])
    ranks_p = jnp.concatenate([ranks, jnp.full((G,), jnp.int32(1 << 30))])

    out = _make_merge_kernel(npad, nblocks)(feats, perm_p, ranks_p, bounds)
    return out[:n]


# speedup vs baseline: 40.3404x; 1.0907x over previous
import functools

import jax
import jax.numpy as jnp
from jax import lax
from jax.experimental import pallas as pl
from jax.experimental.pallas import tpu as pltpu
from jax.experimental.pallas import tpu_sc as plsc

D = 32
W = 16384
G = 128
SP = 512
NI = SP // G
SH = W + 512
TRASH = W
ZR = (SH // 16) // 4
DR = 256


def _extract(bounds_ref, j):
    v = bounds_ref[pl.ds(j, 16)]
    return v[0]


def _make_merge_kernel(npad, nblocks):
    mesh = plsc.VectorSubcoreMesh(core_axis_name="c", subcore_axis_name="s")

    @functools.partial(
        pl.kernel,
        out_type=jax.ShapeDtypeStruct((npad, D), jnp.float32),
        mesh=mesh,
        compiler_params=pltpu.CompilerParams(use_tc_tiling_on_sc=False),
        scratch_types=[
            pltpu.VMEM((80,), jnp.int32),
            pltpu.VMEM((SP,), jnp.int32),
            pltpu.VMEM((SP,), jnp.int32),
            pltpu.VMEM((NI, G), jnp.int32),
            pltpu.VMEM((SP, D), jnp.float32),
            pltpu.VMEM((G, 16), jnp.float32),
            pltpu.VMEM((ZR, D), jnp.float32),
            pltpu.VMEM((ZR, 16), jnp.float32),
            pltpu.VMEM((DR, D), jnp.float32),
            pltpu.VMEM((DR, 16), jnp.float32),
            pltpu.VMEM((DR, D), jnp.float32),
            pltpu.VMEM_SHARED((SH, D), jnp.float32),
            pltpu.VMEM_SHARED((SH, 16), jnp.float32),
            pltpu.SemaphoreType.DMA,
            pltpu.SemaphoreType.DMA,
        ],
    )
    def merge(feat, perm, ranks, bounds, out,
              bounds_v, perm1_v, rank_v, dest2_v, rows_v, ones_v, zrows_v,
              zcnt_v, sstage_v, cstage_v, ostage_v, sums_sh, cnts_sh,
              sem, sem2):
        c = lax.axis_index("c")
        s = lax.axis_index("s")

        pltpu.sync_copy(bounds, bounds_v)

        ones16 = jnp.ones((16,), jnp.float32)
        zeros16 = jnp.zeros((16,), jnp.float32)

        def init_ones(i, carry):
            ones_v[i] = ones16
            return carry
        lax.fori_loop(jnp.int32(0), jnp.int32(G), init_ones, 0)

        def init_zeros(i, carry):
            zcnt_v[i] = zeros16
            zrows_v[i, pl.ds(0, 16)] = zeros16
            zrows_v[i, pl.ds(16, 16)] = zeros16
            return carry
        lax.fori_loop(jnp.int32(0), jnp.int32(ZR), init_zeros, 0)

        nblk = (nblocks - c + 1) // 2

        def block_body(jb, carry):
            k = c + 2 * jb
            base = k * W

            zbase = s * (SH // 16)
            for zi in range(4):
                r0 = zbase + zi * ZR
                pltpu.async_copy(zrows_v, sums_sh.at[pl.ds(r0, ZR)], sem2)
                pltpu.async_copy(zcnt_v, cnts_sh.at[pl.ds(r0, ZR)], sem2)
            for zi in range(4):
                pltpu.make_async_copy(zrows_v, sums_sh.at[pl.ds(zbase, ZR)], sem2).wait()
                pltpu.make_async_copy(zcnt_v, cnts_sh.at[pl.ds(zbase, ZR)], sem2).wait()
            plsc.subcore_barrier()

            lo = _extract(bounds_v, k)
            hi = _extract(bounds_v, k + 1)
            lo8 = (lo // 8) * 8
            nch = (hi - lo8 + (SP - 1)) // SP
            my = jnp.maximum((nch - s + 15) // 16, 0)

            def acc_body(i, acarry):
                m = s + i * 16
                start = lo8 + m * SP
                pltpu.async_copy(perm.at[pl.ds(start, SP)], perm1_v, sem2)
                pltpu.sync_copy(ranks.at[pl.ds(start, SP)], rank_v)
                for t in range(SP // 16):
                    r = rank_v[pl.ds(t * 16, 16)]
                    off = r - base
                    valid = (off >= 0) & (off < W)
                    dest2_v[t // 8, pl.ds((t % 8) * 16, 16)] = \
                        jnp.where(valid, off, TRASH)
                pltpu.make_async_copy(perm.at[pl.ds(start, SP)],
                                      perm1_v, sem2).wait()
                for q in range(NI):
                    pltpu.async_copy(feat.at[perm1_v.at[pl.ds(q * G, G)]],
                                     rows_v.at[pl.ds(q * G, G)], sem)
                for q in range(NI):
                    pltpu.make_async_copy(feat.at[perm1_v.at[pl.ds(q * G, G)]],
                                          rows_v.at[pl.ds(q * G, G)], sem).wait()
                for q in range(NI):
                    pltpu.sync_copy(rows_v.at[pl.ds(q * G, G)],
                                    sums_sh.at[dest2_v.at[jnp.int32(q)]], add=True)
                    pltpu.sync_copy(ones_v,
                                    cnts_sh.at[dest2_v.at[jnp.int32(q)]], add=True)
                return acarry
            lax.fori_loop(jnp.int32(0), my, acc_body, 0)
            plsc.subcore_barrier()

            tbase = s * (W // 16)
            for sub in range((W // 16) // DR):
                r0 = tbase + sub * DR
                pltpu.sync_copy(sums_sh.at[pl.ds(r0, DR)], sstage_v)
                pltpu.sync_copy(cnts_sh.at[pl.ds(r0, DR)], cstage_v)

                def div_body(g, dcarry):
                    for u in range(4):
                        rr = g * 4 + u
                        d = jnp.maximum(cstage_v[rr], 1.0)
                        ostage_v[rr, pl.ds(0, 16)] = sstage_v[rr, pl.ds(0, 16)] / d
                        ostage_v[rr, pl.ds(16, 16)] = sstage_v[rr, pl.ds(16, 16)] / d
                    return dcarry
                lax.fori_loop(jnp.int32(0), jnp.int32(DR // 4), div_body, 0)
                pltpu.sync_copy(ostage_v, out.at[pl.ds(base + r0, DR)])
            plsc.subcore_barrier()
            return carry

        lax.fori_loop(jnp.int32(0), nblk, block_body, 0)

    return merge


def kernel(input1_features, input1_coords, input2_features, input2_coords):
    S = jnp.int64(4096)
    batch_size = 1 + jnp.max(input1_coords[:, 3])
    b2 = input2_coords[:, 3] + batch_size
    k1 = ((input1_coords[:, 3] * S + input1_coords[:, 0]) * S
          + input1_coords[:, 1]) * S + input1_coords[:, 2]
    k2 = ((b2 * S + input2_coords[:, 0]) * S
          + input2_coords[:, 1]) * S + input2_coords[:, 2]
    keys = jnp.concatenate([k1, k2])
    feats = jnp.concatenate([input1_features, input2_features], axis=0)
    n = keys.shape[0]

    sk, perm = lax.sort_key_val(keys, jnp.arange(n, dtype=jnp.int32))
    neq = (sk[1:] != sk[:-1]).astype(jnp.int32)
    ranks = jnp.concatenate([jnp.zeros((1,), jnp.int32), jnp.cumsum(neq, dtype=jnp.int32)])

    nblocks = -(-n // W)
    npad = nblocks * W
    targets = jnp.arange(nblocks + 1, dtype=jnp.int32) * W
    bounds = jnp.searchsorted(ranks, targets, side='left').astype(jnp.int32)
    bounds = jnp.concatenate(
        [bounds, jnp.full((80 - (nblocks + 1),), jnp.int32(n))])

    perm_p = jnp.concatenate([perm, jnp.zeros((SP,), jnp.int32)])
    ranks_p = jnp.concatenate([ranks, jnp.full((SP,), jnp.int32(1 << 30))])

    out = _make_merge_kernel(npad, nblocks)(feats, perm_p, ranks_p, bounds)
    return out[:n]
